# Initial kernel scaffold; baseline (speedup 1.0000x reference)
import jax
import jax.numpy as jnp
from jax.experimental import pallas as pl


def kernel(hidden, inputs, W, b):
    raise NotImplementedError("write your pallas kernel here")


# speedup vs baseline: 2.1182x; 2.1182x over previous
import jax
import jax.numpy as jnp
from jax import lax
from jax.experimental import pallas as pl
from jax.experimental.pallas import tpu as pltpu
from jax.experimental.pallas import tpu_sc as plsc

_N_EMBD = 768
_N_CLASS = 10
_CLF_TOKEN = 40480
_TOTAL = 8192
_NUM_TILES = 32
_ROWS_PER_TILE = _TOTAL // _NUM_TILES
_OUT_PER_TILE = _ROWS_PER_TILE * _N_CLASS
_CHUNKS = _TOTAL // 16
_EMBD_CHUNKS = _N_EMBD // 16


def _sc_body(flat_hbm, hid_hbm, wt_hbm, b_hbm, out_hbm,
             flat_v, idx_v, wt_v, b_v, rows_v, out_v, sem):
    cid = lax.axis_index("c")
    sid = lax.axis_index("s")
    wid = sid * 2 + cid
    base = wid * _ROWS_PER_TILE

    pltpu.sync_copy(flat_hbm, flat_v)
    pltpu.sync_copy(wt_hbm, wt_v)
    pltpu.sync_copy(b_hbm, b_v)

    zero16 = jnp.zeros((16,), jnp.float32)

    def zero_body(i, carry):
        out_v[pl.ds(i * 16, 16)] = zero16
        return carry

    lax.fori_loop(0, _OUT_PER_TILE // 16, zero_body, 0)

    lane = lax.iota(jnp.int32, 16)

    def scan_body(i, off):
        v = flat_v[pl.ds(i * 16, 16)]
        mi = (v == _CLF_TOKEN).astype(jnp.int32)
        ranks = off + plsc.cumsum(mi) - 1
        pos = jnp.where(mi > 0, ranks, _TOTAL + lane)
        plsc.store_scatter(idx_v, [pos], lane + i * 16)
        return off + jnp.sum(mi)

    count = lax.fori_loop(0, _CHUNKS, scan_body, 0)

    b_vec = b_v[...]
    out_mask = lane < _N_CLASS

    n_mine = jnp.clip(count - base, 0, _ROWS_PER_TILE)
    nchunks = (n_mine + 15) // 16

    def chunk_body(k, carry):
        idx16 = jnp.clip(idx_v[pl.ds(base + k * 16, 16)], 0, _TOTAL - 1)
        pltpu.async_copy(hid_hbm.at[idx16], rows_v, sem).wait()
        nrows = jnp.minimum(n_mine - k * 16, 16)

        def row_body(r, carry2):
            def dot_body(j, accs):
                h = rows_v[r, pl.ds(j * 16, 16)]
                return tuple(accs[c] + h * wt_v[c, pl.ds(j * 16, 16)]
                             for c in range(_N_CLASS))

            accs = lax.fori_loop(0, _EMBD_CHUNKS, dot_body,
                                 tuple(zero16 for _ in range(_N_CLASS)))
            logits = b_vec
            for c in range(_N_CLASS):
                logits = jnp.where(lane == c, jnp.sum(accs[c]) + logits, logits)
            rg = k * 16 + r
            pos = jnp.where(out_mask, rg * _N_CLASS + lane, _OUT_PER_TILE + lane)
            plsc.store_scatter(out_v, [pos], logits)
            return carry2

        lax.fori_loop(0, nrows, row_body, 0)
        return carry

    lax.fori_loop(0, nchunks, chunk_body, 0)

    pltpu.sync_copy(out_v.at[pl.ds(0, _OUT_PER_TILE)],
                    out_hbm.at[pl.ds(wid * _OUT_PER_TILE, _OUT_PER_TILE)])


def kernel(hidden, inputs, W, b):
    flat = inputs[..., 0].reshape(-1).astype(jnp.int32)
    hid2d = hidden.reshape(_TOTAL, _N_EMBD)
    wt = W.T.astype(jnp.float32)
    bpad = jnp.zeros((16,), jnp.float32).at[:_N_CLASS].set(b)

    mesh = plsc.VectorSubcoreMesh(core_axis_name="c", subcore_axis_name="s",
                                  num_cores=2, num_subcores=16)
    out = pl.kernel(
        _sc_body,
        out_type=jax.ShapeDtypeStruct((_TOTAL * _N_CLASS,), jnp.float32),
        mesh=mesh,
        compiler_params=pltpu.CompilerParams(needs_layout_passes=False),
        scratch_types=[
            pltpu.VMEM((_TOTAL,), jnp.int32),
            pltpu.VMEM((_TOTAL + 16,), jnp.int32),
            pltpu.VMEM((_N_CLASS, _N_EMBD), jnp.float32),
            pltpu.VMEM((16,), jnp.float32),
            pltpu.VMEM((16, _N_EMBD), jnp.float32),
            pltpu.VMEM((_OUT_PER_TILE + 16,), jnp.float32),
            pltpu.SemaphoreType.DMA,
        ],
    )(flat, hid2d, wt, bpad)
    return out.reshape(_TOTAL, _N_CLASS)


# speedup vs baseline: 2.4242x; 1.1444x over previous
import jax
import jax.numpy as jnp
from jax import lax
from jax.experimental import pallas as pl
from jax.experimental.pallas import tpu as pltpu
from jax.experimental.pallas import tpu_sc as plsc

_N_EMBD = 768
_N_CLASS = 10
_CLF_TOKEN = 40480
_TOTAL = 8192
_NUM_TILES = 32
_ROWS_PER_TILE = _TOTAL // _NUM_TILES
_OUT_PER_TILE = _ROWS_PER_TILE * _N_CLASS
_CHUNKS = _TOTAL // 16
_GROUP = 16
_EMBD_CHUNKS = _N_EMBD // 16


def _sc_body(flat_hbm, hid_hbm, wt_hbm, b_hbm, out_hbm,
             flat_v, idx_v, wt_v, b_v, rows_v, out_v, sem):
    cid = lax.axis_index("c")
    sid = lax.axis_index("s")
    wid = sid * 2 + cid
    base = wid * _ROWS_PER_TILE

    pltpu.sync_copy(flat_hbm, flat_v)
    pltpu.sync_copy(wt_hbm, wt_v)
    pltpu.sync_copy(b_hbm, b_v)

    zero16 = jnp.zeros((16,), jnp.float32)

    def zero_body(i, carry):
        out_v[pl.ds(i * 16, 16)] = zero16
        return carry

    lax.fori_loop(0, _OUT_PER_TILE // 16, zero_body, 0)

    lane = lax.iota(jnp.int32, 16)

    def scan_chunk(i, off):
        v = flat_v[pl.ds(i * 16, 16)]
        mi = (v == _CLF_TOKEN).astype(jnp.int32)
        ranks = off + plsc.cumsum(mi) - 1
        pos = jnp.where(mi > 0, ranks, _TOTAL + lane)
        plsc.store_scatter(idx_v, [pos], lane + i * 16)
        return off + jnp.sum(mi)

    def group_body(g, off):
        acc = jnp.zeros((16,), jnp.int32)
        for t in range(_GROUP):
            v = flat_v[pl.ds(g * (16 * _GROUP) + t * 16, 16)]
            acc = acc | (v == _CLF_TOKEN).astype(jnp.int32)

        def rare(o):
            return lax.fori_loop(g * _GROUP, (g + 1) * _GROUP, scan_chunk, o)

        return lax.cond(jnp.sum(acc) > 0, rare, lambda o: o, off)

    count = lax.fori_loop(0, _CHUNKS // _GROUP, group_body, 0)

    b_vec = b_v[...]
    out_mask = lane < _N_CLASS

    n_mine = jnp.clip(count - base, 0, _ROWS_PER_TILE)
    nchunks = (n_mine + 15) // 16

    def chunk_body(k, carry):
        idx16 = jnp.clip(idx_v[pl.ds(base + k * 16, 16)], 0, _TOTAL - 1)
        pltpu.async_copy(hid_hbm.at[idx16], rows_v, sem).wait()
        nrows = jnp.minimum(n_mine - k * 16, 16)

        def row_body(r, carry2):
            def dot_body(j, accs):
                h = rows_v[r, pl.ds(j * 16, 16)]
                return tuple(accs[c] + h * wt_v[c, pl.ds(j * 16, 16)]
                             for c in range(_N_CLASS))

            accs = lax.fori_loop(0, _EMBD_CHUNKS, dot_body,
                                 tuple(zero16 for _ in range(_N_CLASS)))
            logits = b_vec
            for c in range(_N_CLASS):
                logits = jnp.where(lane == c, jnp.sum(accs[c]) + logits, logits)
            rg = k * 16 + r
            pos = jnp.where(out_mask, rg * _N_CLASS + lane, _OUT_PER_TILE + lane)
            plsc.store_scatter(out_v, [pos], logits)
            return carry2

        lax.fori_loop(0, nrows, row_body, 0)
        return carry

    lax.fori_loop(0, nchunks, chunk_body, 0)

    pltpu.sync_copy(out_v.at[pl.ds(0, _OUT_PER_TILE)],
                    out_hbm.at[pl.ds(wid * _OUT_PER_TILE, _OUT_PER_TILE)])


def kernel(hidden, inputs, W, b):
    flat = inputs[..., 0].reshape(-1).astype(jnp.int32)
    hid2d = hidden.reshape(_TOTAL, _N_EMBD)
    wt = W.T.astype(jnp.float32)
    bpad = jnp.zeros((16,), jnp.float32).at[:_N_CLASS].set(b)

    mesh = plsc.VectorSubcoreMesh(core_axis_name="c", subcore_axis_name="s",
                                  num_cores=2, num_subcores=16)
    out = pl.kernel(
        _sc_body,
        out_type=jax.ShapeDtypeStruct((_TOTAL * _N_CLASS,), jnp.float32),
        mesh=mesh,
        compiler_params=pltpu.CompilerParams(needs_layout_passes=False),
        scratch_types=[
            pltpu.VMEM((_TOTAL,), jnp.int32),
            pltpu.VMEM((_TOTAL + 16,), jnp.int32),
            pltpu.VMEM((_N_CLASS, _N_EMBD), jnp.float32),
            pltpu.VMEM((16,), jnp.float32),
            pltpu.VMEM((16, _N_EMBD), jnp.float32),
            pltpu.VMEM((_OUT_PER_TILE + 16,), jnp.float32),
            pltpu.SemaphoreType.DMA,
        ],
    )(flat, hid2d, wt, bpad)
    return out.reshape(_TOTAL, _N_CLASS)


# speedup vs baseline: 2.7731x; 1.1439x over previous
import jax
import jax.numpy as jnp
from jax import lax
from jax.experimental import pallas as pl
from jax.experimental.pallas import tpu as pltpu
from jax.experimental.pallas import tpu_sc as plsc

_N_EMBD = 768
_N_CLASS = 10
_CLF_TOKEN = 40480
_TOTAL = 8192
_NUM_TILES = 32
_ROWS_PER_TILE = _TOTAL // _NUM_TILES
_OUT_PER_TILE = _ROWS_PER_TILE * _N_CLASS
_CHUNKS = _TOTAL // 16
_GROUP = 16
_EMBD_CHUNKS = _N_EMBD // 16


def _sc_body(flat_hbm, hid_hbm, wt_hbm, b_hbm, out_hbm,
             flat_v, idx_v, wt_v, b_v, rows_v, out_v, sem):
    cid = lax.axis_index("c")
    sid = lax.axis_index("s")
    wid = sid * 2 + cid
    base = wid * _ROWS_PER_TILE

    flat_dma = pltpu.async_copy(flat_hbm, flat_v, sem)

    zero16 = jnp.zeros((16,), jnp.float32)

    def zero_body(i, carry):
        out_v[pl.ds(i * 16, 16)] = zero16
        return carry

    lax.fori_loop(0, _OUT_PER_TILE // 16, zero_body, 0)

    flat_dma.wait()

    lane = lax.iota(jnp.int32, 16)
    zi32 = jnp.zeros((16,), jnp.int32)

    def det_body(i, accs):
        return tuple(jnp.maximum(accs[u], flat_v[pl.ds(i * 128 + u * 16, 16)])
                     for u in range(8))

    daccs = lax.fori_loop(0, _CHUNKS // 8, det_body, (zi32,) * 8)
    dm = jnp.maximum(jnp.maximum(jnp.maximum(daccs[0], daccs[1]),
                                 jnp.maximum(daccs[2], daccs[3])),
                     jnp.maximum(jnp.maximum(daccs[4], daccs[5]),
                                 jnp.maximum(daccs[6], daccs[7])))
    any_match = jnp.max(dm) >= _CLF_TOKEN

    def scan_chunk(i, off):
        v = flat_v[pl.ds(i * 16, 16)]
        mi = (v == _CLF_TOKEN).astype(jnp.int32)
        ranks = off + plsc.cumsum(mi) - 1
        pos = jnp.where(mi > 0, ranks, _TOTAL + lane)
        plsc.store_scatter(idx_v, [pos], lane + i * 16)
        return off + jnp.sum(mi)

    def group_body(g, off):
        acc = jnp.zeros((16,), jnp.int32)
        for t in range(_GROUP):
            v = flat_v[pl.ds(g * (16 * _GROUP) + t * 16, 16)]
            acc = acc | (v == _CLF_TOKEN).astype(jnp.int32)

        def rare(o):
            return lax.fori_loop(g * _GROUP, (g + 1) * _GROUP, scan_chunk, o)

        return lax.cond(jnp.sum(acc) > 0, rare, lambda o: o, off)

    @pl.when(any_match)
    def _rare_path():
        count = lax.fori_loop(0, _CHUNKS // _GROUP, group_body, 0)

        pltpu.sync_copy(wt_hbm, wt_v)
        pltpu.sync_copy(b_hbm, b_v)
        b_vec = b_v[...]
        out_mask = lane < _N_CLASS

        n_mine = jnp.clip(count - base, 0, _ROWS_PER_TILE)
        nchunks = (n_mine + 15) // 16

        def chunk_body(k, carry):
            idx16 = jnp.clip(idx_v[pl.ds(base + k * 16, 16)], 0, _TOTAL - 1)
            pltpu.async_copy(hid_hbm.at[idx16], rows_v, sem).wait()
            nrows = jnp.minimum(n_mine - k * 16, 16)

            def row_body(r, carry2):
                def dot_body(j, accs):
                    h = rows_v[r, pl.ds(j * 16, 16)]
                    return tuple(accs[c] + h * wt_v[c, pl.ds(j * 16, 16)]
                                 for c in range(_N_CLASS))

                accs = lax.fori_loop(0, _EMBD_CHUNKS, dot_body,
                                     tuple(zero16 for _ in range(_N_CLASS)))
                logits = b_vec
                for c in range(_N_CLASS):
                    logits = jnp.where(lane == c,
                                       jnp.sum(accs[c]) + logits, logits)
                rg = k * 16 + r
                pos = jnp.where(out_mask, rg * _N_CLASS + lane,
                                _OUT_PER_TILE + lane)
                plsc.store_scatter(out_v, [pos], logits)
                return carry2

            lax.fori_loop(0, nrows, row_body, 0)
            return carry

        lax.fori_loop(0, nchunks, chunk_body, 0)

    pltpu.sync_copy(out_v.at[pl.ds(0, _OUT_PER_TILE)],
                    out_hbm.at[pl.ds(wid * _OUT_PER_TILE, _OUT_PER_TILE)])


def kernel(hidden, inputs, W, b):
    flat = inputs[..., 0].reshape(-1).astype(jnp.int32)
    hid2d = hidden.reshape(_TOTAL, _N_EMBD)
    wt = W.T.astype(jnp.float32)
    bpad = jnp.zeros((16,), jnp.float32).at[:_N_CLASS].set(b)

    mesh = plsc.VectorSubcoreMesh(core_axis_name="c", subcore_axis_name="s",
                                  num_cores=2, num_subcores=16)
    out = pl.kernel(
        _sc_body,
        out_type=jax.ShapeDtypeStruct((_TOTAL * _N_CLASS,), jnp.float32),
        mesh=mesh,
        compiler_params=pltpu.CompilerParams(needs_layout_passes=False),
        scratch_types=[
            pltpu.VMEM((_TOTAL,), jnp.int32),
            pltpu.VMEM((_TOTAL + 16,), jnp.int32),
            pltpu.VMEM((_N_CLASS, _N_EMBD), jnp.float32),
            pltpu.VMEM((16,), jnp.float32),
            pltpu.VMEM((16, _N_EMBD), jnp.float32),
            pltpu.VMEM((_OUT_PER_TILE + 16,), jnp.float32),
            pltpu.SemaphoreType.DMA,
        ],
    )(flat, hid2d, wt, bpad)
    return out.reshape(_TOTAL, _N_CLASS)


# speedup vs baseline: 3.2610x; 1.1760x over previous
import jax
import jax.numpy as jnp
from jax import lax
from jax.experimental import pallas as pl
from jax.experimental.pallas import tpu as pltpu
from jax.experimental.pallas import tpu_sc as plsc

_N_EMBD = 768
_N_CLASS = 10
_CLF_TOKEN = 40480
_TOTAL = 8192
_NUM_TILES = 32
_ROWS_PER_TILE = _TOTAL // _NUM_TILES
_CHUNKS = _TOTAL // 16
_GROUP = 16
_EMBD_CHUNKS = _N_EMBD // 16


def _sc_body(flat_hbm, hid_hbm, wt_hbm, b_hbm, lt_hbm, cnt_hbm,
             flat_v, idx_v, wt_v, b_v, rows_v, lout_v, cnt_v, sem):
    cid = lax.axis_index("c")
    sid = lax.axis_index("s")
    wid = sid * 2 + cid
    base = wid * _ROWS_PER_TILE

    pltpu.sync_copy(flat_hbm, flat_v)

    lane = lax.iota(jnp.int32, 16)
    zi32 = jnp.zeros((16,), jnp.int32)
    zero16 = jnp.zeros((16,), jnp.float32)

    def det_body(i, accs):
        return tuple(jnp.maximum(accs[u], flat_v[pl.ds(i * 128 + u * 16, 16)])
                     for u in range(8))

    daccs = lax.fori_loop(0, _CHUNKS // 8, det_body, (zi32,) * 8)
    dm = jnp.maximum(jnp.maximum(jnp.maximum(daccs[0], daccs[1]),
                                 jnp.maximum(daccs[2], daccs[3])),
                     jnp.maximum(jnp.maximum(daccs[4], daccs[5]),
                                 jnp.maximum(daccs[6], daccs[7])))
    any_match = jnp.max(dm) >= _CLF_TOKEN

    def scan_chunk(i, off):
        v = flat_v[pl.ds(i * 16, 16)]
        mi = (v == _CLF_TOKEN).astype(jnp.int32)
        ranks = off + plsc.cumsum(mi) - 1
        pos = jnp.where(mi > 0, ranks, _TOTAL + lane)
        plsc.store_scatter(idx_v, [pos], lane + i * 16)
        return off + jnp.sum(mi)

    def group_body(g, off):
        acc = zi32
        for t in range(_GROUP):
            v = flat_v[pl.ds(g * (16 * _GROUP) + t * 16, 16)]
            acc = acc | (v == _CLF_TOKEN).astype(jnp.int32)

        def rare(o):
            return lax.fori_loop(g * _GROUP, (g + 1) * _GROUP, scan_chunk, o)

        return lax.cond(jnp.sum(acc) > 0, rare, lambda o: o, off)

    count = lax.cond(any_match,
                     lambda: lax.fori_loop(0, _CHUNKS // _GROUP, group_body, 0),
                     lambda: 0)

    @pl.when(wid == 0)
    def _write_count():
        cnt_v[...] = zi32 + count
        pltpu.sync_copy(cnt_v, cnt_hbm)

    n_mine = jnp.clip(count - base, 0, _ROWS_PER_TILE)

    @pl.when(n_mine > 0)
    def _compute_rows():
        pltpu.sync_copy(wt_hbm, wt_v)
        pltpu.sync_copy(b_hbm, b_v)
        b_vec = b_v[...]
        nchunks = (n_mine + 15) // 16

        def chunk_body(k, carry):
            idx16 = jnp.clip(idx_v[pl.ds(base + k * 16, 16)], 0, _TOTAL - 1)
            pltpu.async_copy(hid_hbm.at[idx16], rows_v, sem).wait()
            nrows = jnp.minimum(n_mine - k * 16, 16)

            def row_body(r, carry2):
                def dot_body(j, accs):
                    h = rows_v[r, pl.ds(j * 16, 16)]
                    return tuple(accs[c] + h * wt_v[c, pl.ds(j * 16, 16)]
                                 for c in range(_N_CLASS))

                accs = lax.fori_loop(0, _EMBD_CHUNKS, dot_body,
                                     tuple(zero16 for _ in range(_N_CLASS)))
                logits = b_vec
                for c in range(_N_CLASS):
                    logits = jnp.where(lane == c,
                                       jnp.sum(accs[c]) + logits, logits)
                plsc.store_scatter(lout_v, [lane, zi32 + (k * 16 + r)], logits)
                return carry2

            lax.fori_loop(0, nrows, row_body, 0)
            return carry

        lax.fori_loop(0, nchunks, chunk_body, 0)
        pltpu.sync_copy(lout_v, lt_hbm.at[:, pl.ds(base, _ROWS_PER_TILE)])


def kernel(hidden, inputs, W, b):
    flat = inputs[..., 0].reshape(-1).astype(jnp.int32)
    hid2d = hidden.reshape(_TOTAL, _N_EMBD)
    wt = W.T.astype(jnp.float32)
    bpad = jnp.zeros((16,), jnp.float32).at[:_N_CLASS].set(b)

    mesh = plsc.VectorSubcoreMesh(core_axis_name="c", subcore_axis_name="s",
                                  num_cores=2, num_subcores=16)
    lt, cnt = pl.kernel(
        _sc_body,
        out_type=(jax.ShapeDtypeStruct((16, _TOTAL), jnp.float32),
                  jax.ShapeDtypeStruct((16,), jnp.int32)),
        mesh=mesh,
        compiler_params=pltpu.CompilerParams(needs_layout_passes=False),
        scratch_types=[
            pltpu.VMEM((_TOTAL,), jnp.int32),
            pltpu.VMEM((_TOTAL + 16,), jnp.int32),
            pltpu.VMEM((_N_CLASS, _N_EMBD), jnp.float32),
            pltpu.VMEM((16,), jnp.float32),
            pltpu.VMEM((16, _N_EMBD), jnp.float32),
            pltpu.VMEM((16, _ROWS_PER_TILE), jnp.float32),
            pltpu.VMEM((16,), jnp.int32),
            pltpu.SemaphoreType.DMA,
        ],
    )(flat, hid2d, wt, bpad)
    valid = jnp.arange(_TOTAL, dtype=jnp.int32) < cnt[0]
    return jnp.where(valid[:, None], lt[:_N_CLASS, :].T, jnp.float32(0.0))


# speedup vs baseline: 3.2739x; 1.0040x over previous
import jax
import jax.numpy as jnp
from jax import lax
from jax.experimental import pallas as pl
from jax.experimental.pallas import tpu as pltpu
from jax.experimental.pallas import tpu_sc as plsc

_N_EMBD = 768
_N_CLASS = 10
_CLF_TOKEN = 40480
_TOTAL = 8192
_NUM_TILES = 32
_ROWS_PER_TILE = _TOTAL // _NUM_TILES
_CHUNKS = _TOTAL // 16
_GROUP = 16
_EMBD_CHUNKS = _N_EMBD // 16


def _sc_body(flat_hbm, hid_hbm, wt_hbm, lt_hbm, cnt_hbm,
             flat_v, idx_v, wt_v, rows_v, lout_v, cnt_v, sem):
    cid = lax.axis_index("c")
    sid = lax.axis_index("s")
    wid = sid * 2 + cid
    base = wid * _ROWS_PER_TILE

    pltpu.sync_copy(flat_hbm, flat_v)

    lane = lax.iota(jnp.int32, 16)
    zi32 = jnp.zeros((16,), jnp.int32)
    zero16 = jnp.zeros((16,), jnp.float32)

    def det_body(i, accs):
        return tuple(jnp.maximum(accs[u], flat_v[pl.ds(i * 128 + u * 16, 16)])
                     for u in range(8))

    daccs = lax.fori_loop(0, _CHUNKS // 8, det_body, (zi32,) * 8)
    dm = jnp.maximum(jnp.maximum(jnp.maximum(daccs[0], daccs[1]),
                                 jnp.maximum(daccs[2], daccs[3])),
                     jnp.maximum(jnp.maximum(daccs[4], daccs[5]),
                                 jnp.maximum(daccs[6], daccs[7])))
    any_match = jnp.max(dm) >= _CLF_TOKEN

    def scan_chunk(i, off):
        v = flat_v[pl.ds(i * 16, 16)]
        mi = (v == _CLF_TOKEN).astype(jnp.int32)
        ranks = off + plsc.cumsum(mi) - 1
        pos = jnp.where(mi > 0, ranks, _TOTAL + lane)
        plsc.store_scatter(idx_v, [pos], lane + i * 16)
        return off + jnp.sum(mi)

    count = lax.cond(any_match,
                     lambda: lax.fori_loop(0, _CHUNKS, scan_chunk, 0),
                     lambda: 0)

    @pl.when(wid == 0)
    def _write_count():
        cnt_v[...] = zi32 + count
        pltpu.sync_copy(cnt_v, cnt_hbm)

    n_mine = jnp.clip(count - base, 0, _ROWS_PER_TILE)

    @pl.when(n_mine > 0)
    def _compute_rows():
        pltpu.sync_copy(wt_hbm, wt_v)
        nchunks = (n_mine + 15) // 16

        def chunk_body(k, carry):
            idx16 = jnp.clip(idx_v[pl.ds(base + k * 16, 16)], 0, _TOTAL - 1)
            pltpu.async_copy(hid_hbm.at[idx16], rows_v, sem).wait()
            nrows = jnp.minimum(n_mine - k * 16, 16)

            def row_body(r, carry2):
                def class_body(c, logits):
                    def dot_body(j, acc):
                        return acc + (rows_v[r, pl.ds(j * 16, 16)]
                                      * wt_v[c, pl.ds(j * 16, 16)])

                    acc = lax.fori_loop(0, _EMBD_CHUNKS, dot_body, zero16)
                    return jnp.where(lane == c, jnp.sum(acc), logits)

                logits = lax.fori_loop(0, _N_CLASS, class_body, zero16)
                plsc.store_scatter(lout_v, [lane, zi32 + (k * 16 + r)], logits)
                return carry2

            lax.fori_loop(0, nrows, row_body, 0)
            return carry

        lax.fori_loop(0, nchunks, chunk_body, 0)
        pltpu.sync_copy(lout_v, lt_hbm.at[:, pl.ds(base, _ROWS_PER_TILE)])


def kernel(hidden, inputs, W, b):
    flat = inputs[..., 0].reshape(-1).astype(jnp.int32)
    hid2d = hidden.reshape(_TOTAL, _N_EMBD)
    wt = W.T.astype(jnp.float32)

    mesh = plsc.VectorSubcoreMesh(core_axis_name="c", subcore_axis_name="s",
                                  num_cores=2, num_subcores=16)
    lt, cnt = pl.kernel(
        _sc_body,
        out_type=(jax.ShapeDtypeStruct((16, _TOTAL), jnp.float32),
                  jax.ShapeDtypeStruct((16,), jnp.int32)),
        mesh=mesh,
        compiler_params=pltpu.CompilerParams(needs_layout_passes=False),
        scratch_types=[
            pltpu.VMEM((_TOTAL,), jnp.int32),
            pltpu.VMEM((_TOTAL + 16,), jnp.int32),
            pltpu.VMEM((_N_CLASS, _N_EMBD), jnp.float32),
            pltpu.VMEM((16, _N_EMBD), jnp.float32),
            pltpu.VMEM((16, _ROWS_PER_TILE), jnp.float32),
            pltpu.VMEM((16,), jnp.int32),
            pltpu.SemaphoreType.DMA,
        ],
    )(flat, hid2d, wt)
    valid = jnp.arange(_TOTAL, dtype=jnp.int32)[:, None] < cnt[0]
    return jnp.where(valid, lt[:_N_CLASS, :].T + b[None, :], jnp.float32(0.0))


# speedup vs baseline: 4.5302x; 1.3837x over previous
import jax
import jax.numpy as jnp
from jax import lax
from jax.experimental import pallas as pl
from jax.experimental.pallas import tpu as pltpu
from jax.experimental.pallas import tpu_sc as plsc

_N_EMBD = 768
_N_CLASS = 10
_CLF_TOKEN = 40480
_TOTAL = 8192
_NUM_TILES = 32
_ROWS_PER_TILE = _TOTAL // _NUM_TILES
_CHUNKS = _TOTAL // 16
_EMBD_CHUNKS = _N_EMBD // 16


def _detect_body(flat_ref, mx_ref):
    mx_ref[0, 0] = jnp.max(flat_ref[...])


def _sc_body(flat_hbm, hid_hbm, wt_hbm, lt_hbm, cnt_hbm,
             flat_v, idx_v, wt_v, rows_v, lout_v, cnt_v, sem):
    cid = lax.axis_index("c")
    sid = lax.axis_index("s")
    wid = sid * 2 + cid
    base = wid * _ROWS_PER_TILE

    pltpu.sync_copy(flat_hbm, flat_v)

    lane = lax.iota(jnp.int32, 16)
    zi32 = jnp.zeros((16,), jnp.int32)
    zero16 = jnp.zeros((16,), jnp.float32)

    def scan_chunk(i, off):
        v = flat_v[pl.ds(i * 16, 16)]
        mi = (v == _CLF_TOKEN).astype(jnp.int32)
        ranks = off + plsc.cumsum(mi) - 1
        pos = jnp.where(mi > 0, ranks, _TOTAL + lane)
        plsc.store_scatter(idx_v, [pos], lane + i * 16)
        return off + jnp.sum(mi)

    count = lax.fori_loop(0, _CHUNKS, scan_chunk, 0)

    @pl.when(wid == 0)
    def _write_count():
        cnt_v[...] = zi32 + count
        pltpu.sync_copy(cnt_v, cnt_hbm)

    n_mine = jnp.clip(count - base, 0, _ROWS_PER_TILE)

    @pl.when(n_mine > 0)
    def _compute_rows():
        pltpu.sync_copy(wt_hbm, wt_v)
        nchunks = (n_mine + 15) // 16

        def chunk_body(k, carry):
            idx16 = jnp.clip(idx_v[pl.ds(base + k * 16, 16)], 0, _TOTAL - 1)
            pltpu.async_copy(hid_hbm.at[idx16], rows_v, sem).wait()
            nrows = jnp.minimum(n_mine - k * 16, 16)

            def row_body(r, carry2):
                def class_body(c, logits):
                    def dot_body(j, acc):
                        return acc + (rows_v[r, pl.ds(j * 16, 16)]
                                      * wt_v[c, pl.ds(j * 16, 16)])

                    acc = lax.fori_loop(0, _EMBD_CHUNKS, dot_body, zero16)
                    return jnp.where(lane == c, jnp.sum(acc), logits)

                logits = lax.fori_loop(0, _N_CLASS, class_body, zero16)
                plsc.store_scatter(lout_v, [lane, zi32 + (k * 16 + r)], logits)
                return carry2

            lax.fori_loop(0, nrows, row_body, 0)
            return carry

        lax.fori_loop(0, nchunks, chunk_body, 0)
        pltpu.sync_copy(lout_v, lt_hbm.at[:, pl.ds(base, _ROWS_PER_TILE)])


def kernel(hidden, inputs, W, b):
    flat = inputs[..., 0].reshape(-1).astype(jnp.int32)
    hid2d = hidden.reshape(_TOTAL, _N_EMBD)
    wt = W.T.astype(jnp.float32)

    mx = pl.pallas_call(
        _detect_body,
        out_shape=jax.ShapeDtypeStruct((1, 1), jnp.int32),
        in_specs=[pl.BlockSpec(memory_space=pltpu.VMEM)],
        out_specs=pl.BlockSpec(memory_space=pltpu.SMEM),
    )(flat.reshape(64, 128))
    any_match = mx[0, 0] >= _CLF_TOKEN

    def with_matches():
        mesh = plsc.VectorSubcoreMesh(core_axis_name="c", subcore_axis_name="s",
                                      num_cores=2, num_subcores=16)
        lt, cnt = pl.kernel(
            _sc_body,
            out_type=(jax.ShapeDtypeStruct((16, _TOTAL), jnp.float32),
                      jax.ShapeDtypeStruct((16,), jnp.int32)),
            mesh=mesh,
            compiler_params=pltpu.CompilerParams(needs_layout_passes=False),
            scratch_types=[
                pltpu.VMEM((_TOTAL,), jnp.int32),
                pltpu.VMEM((_TOTAL + 16,), jnp.int32),
                pltpu.VMEM((_N_CLASS, _N_EMBD), jnp.float32),
                pltpu.VMEM((16, _N_EMBD), jnp.float32),
                pltpu.VMEM((16, _ROWS_PER_TILE), jnp.float32),
                pltpu.VMEM((16,), jnp.int32),
                pltpu.SemaphoreType.DMA,
            ],
        )(flat, hid2d, wt)
        valid = jnp.arange(_TOTAL, dtype=jnp.int32)[:, None] < cnt[0]
        return jnp.where(valid, lt[:_N_CLASS, :].T + b[None, :],
                         jnp.float32(0.0))

    def no_matches():
        return jnp.zeros((_TOTAL, _N_CLASS), jnp.float32)

    return lax.cond(any_match, with_matches, no_matches)


# speedup vs baseline: 4.9545x; 1.0937x over previous
import jax
import jax.numpy as jnp
from jax import lax
from jax.experimental import pallas as pl
from jax.experimental.pallas import tpu as pltpu
from jax.experimental.pallas import tpu_sc as plsc

_N_EMBD = 768
_N_CLASS = 10
_CLF_TOKEN = 40480
_TOTAL = 8192
_NUM_TILES = 16
_ROWS_PER_TILE = _TOTAL // _NUM_TILES
_CHUNKS = _TOTAL // 16
_EMBD_CHUNKS = _N_EMBD // 16


def _detect_body(flat_ref, mx_ref):
    mx_ref[0, 0] = jnp.max(flat_ref[...])


def _sc_body(flat_hbm, hid_hbm, wt_hbm, lt_hbm, cnt_hbm,
             flat_v, idx_v, wt_v, rows_v, lout_v, cnt_v, sem):
    wid = lax.axis_index("s")
    base = wid * _ROWS_PER_TILE

    pltpu.sync_copy(flat_hbm, flat_v)

    lane = lax.iota(jnp.int32, 16)
    zi32 = jnp.zeros((16,), jnp.int32)
    zero16 = jnp.zeros((16,), jnp.float32)

    def scan_chunk(i, off):
        v = flat_v[pl.ds(i * 16, 16)]
        mi = (v == _CLF_TOKEN).astype(jnp.int32)
        ranks = off + plsc.cumsum(mi) - 1
        pos = jnp.where(mi > 0, ranks, _TOTAL + lane)
        plsc.store_scatter(idx_v, [pos], lane + i * 16)
        return off + jnp.sum(mi)

    count = lax.fori_loop(0, _CHUNKS, scan_chunk, 0)

    @pl.when(wid == 0)
    def _write_count():
        cnt_v[...] = zi32 + count
        pltpu.sync_copy(cnt_v, cnt_hbm)

    n_mine = jnp.clip(count - base, 0, _ROWS_PER_TILE)

    @pl.when(n_mine > 0)
    def _compute_rows():
        pltpu.sync_copy(wt_hbm, wt_v)
        nchunks = (n_mine + 15) // 16

        def chunk_body(k, carry):
            idx16 = jnp.clip(idx_v[pl.ds(base + k * 16, 16)], 0, _TOTAL - 1)
            pltpu.async_copy(hid_hbm.at[idx16], rows_v, sem).wait()
            nrows = jnp.minimum(n_mine - k * 16, 16)

            def row_body(r, carry2):
                def class_body(c, logits):
                    def dot_body(j, acc):
                        return acc + (rows_v[r, pl.ds(j * 16, 16)]
                                      * wt_v[c, pl.ds(j * 16, 16)])

                    acc = lax.fori_loop(0, _EMBD_CHUNKS, dot_body, zero16)
                    return jnp.where(lane == c, jnp.sum(acc), logits)

                logits = lax.fori_loop(0, _N_CLASS, class_body, zero16)
                plsc.store_scatter(lout_v, [lane, zi32 + (k * 16 + r)], logits)
                return carry2

            lax.fori_loop(0, nrows, row_body, 0)
            return carry

        lax.fori_loop(0, nchunks, chunk_body, 0)
        pltpu.sync_copy(lout_v, lt_hbm.at[:, pl.ds(base, _ROWS_PER_TILE)])


def kernel(hidden, inputs, W, b):
    flat = inputs[..., 0].reshape(-1).astype(jnp.int32)
    hid2d = hidden.reshape(_TOTAL, _N_EMBD)
    wt = W.T.astype(jnp.float32)

    mx = pl.pallas_call(
        _detect_body,
        out_shape=jax.ShapeDtypeStruct((1, 1), jnp.int32),
        in_specs=[pl.BlockSpec(memory_space=pltpu.VMEM)],
        out_specs=pl.BlockSpec(memory_space=pltpu.SMEM),
    )(flat.reshape(64, 128))
    any_match = mx[0, 0] >= _CLF_TOKEN

    def with_matches():
        mesh = plsc.VectorSubcoreMesh(core_axis_name="c", subcore_axis_name="s",
                                      num_cores=1, num_subcores=16)
        lt, cnt = pl.kernel(
            _sc_body,
            out_type=(jax.ShapeDtypeStruct((16, _TOTAL), jnp.float32),
                      jax.ShapeDtypeStruct((16,), jnp.int32)),
            mesh=mesh,
            compiler_params=pltpu.CompilerParams(needs_layout_passes=False),
            scratch_types=[
                pltpu.VMEM((_TOTAL,), jnp.int32),
                pltpu.VMEM((_TOTAL + 16,), jnp.int32),
                pltpu.VMEM((_N_CLASS, _N_EMBD), jnp.float32),
                pltpu.VMEM((16, _N_EMBD), jnp.float32),
                pltpu.VMEM((16, _ROWS_PER_TILE), jnp.float32),
                pltpu.VMEM((16,), jnp.int32),
                pltpu.SemaphoreType.DMA,
            ],
        )(flat, hid2d, wt)
        valid = jnp.arange(_TOTAL, dtype=jnp.int32)[:, None] < cnt[0]
        return jnp.where(valid, lt[:_N_CLASS, :].T + b[None, :],
                         jnp.float32(0.0))

    def no_matches():
        return jnp.zeros((_TOTAL, _N_CLASS), jnp.float32)

    return lax.cond(any_match, with_matches, no_matches)
